# baseline (device time: 166627 ns/iter reference)
import jax
import jax.numpy as jnp
from jax import lax
from jax.experimental import pallas as pl
from jax.experimental.pallas import tpu as pltpu

N_DEV = 16
B_PER = 256
D = 256
H_PER = 512
GROUP = 4
BF16 = jnp.bfloat16
F32 = jnp.float32
MESH = pl.DeviceIdType.MESH


def kernel(x, Win0, Wout0, Win1, Wout1, Win2, Wout2):
    def body(x_ref, win0_ref, wout0_ref, win1_ref, wout1_ref, win2_ref,
             wout2_ref, out_ref, X_ref, Pb_ref, rs_ref,
             ssem_ag, ssem_rs, rsem_ag, rsem_rs):
        my = lax.axis_index("i")

        def slot(k, n=1):
            return pl.ds(k * B_PER, n * B_PER)

        bsem = pltpu.get_barrier_semaphore()
        for k in range(1, N_DEV):
            pl.semaphore_signal(bsem, inc=1, device_id=((my + k) % N_DEV,),
                                device_id_type=MESH)
        pl.semaphore_wait(bsem, N_DEV - 1)

        def broadcast_own_chunk():
            sends = []
            for k in range(1, N_DEV):
                d = pltpu.make_async_remote_copy(
                    src_ref=X_ref.at[slot(0)],
                    dst_ref=X_ref.at[slot(k)],
                    send_sem=ssem_ag.at[k - 1], recv_sem=rsem_ag.at[k - 1],
                    device_id=((my + k) % N_DEV,), device_id_type=MESH)
                d.start()
                sends.append(d)
            return sends

        w_in = (win0_ref, win1_ref, win2_ref)
        w_out = (wout0_ref, wout1_ref, wout2_ref)

        X_ref[slot(0), :] = x_ref[:].astype(BF16)
        ag_sends = broadcast_own_chunk()

        for L in range(3):
            Wi = w_in[L][:].astype(BF16)
            Wo = w_out[L][:].astype(BF16)
            rs_sends = []
            p_own = None
            for g in range(N_DEV // GROUP):
                for k in range(g * GROUP, (g + 1) * GROUP):
                    if k == 0:
                        continue
                    recv = pltpu.make_async_remote_copy(
                        src_ref=X_ref.at[slot(k)],
                        dst_ref=X_ref.at[slot(k)],
                        send_sem=ssem_ag.at[k - 1],
                        recv_sem=rsem_ag.at[k - 1],
                        device_id=((my - k) % N_DEV,),
                        device_id_type=MESH)
                    recv.wait_recv()
                Xg = X_ref[slot(g * GROUP, GROUP), :]
                Hg = jnp.maximum(
                    jnp.dot(Xg, Wi, preferred_element_type=F32), 0.0)
                Pg = jnp.dot(Hg.astype(BF16), Wo,
                             preferred_element_type=F32)
                for j in range(GROUP):
                    k = g * GROUP + j
                    if k == 0:
                        p_own = Pg[:B_PER, :]
                        continue
                    Pb_ref[slot(k), :] = Pg[
                        j * B_PER:(j + 1) * B_PER, :].astype(BF16)
                    d = pltpu.make_async_remote_copy(
                        src_ref=Pb_ref.at[slot(k)],
                        dst_ref=rs_ref.at[15 - k],
                        send_sem=ssem_rs.at[15 - k],
                        recv_sem=rsem_rs.at[15 - k],
                        device_id=((my - k) % N_DEV,),
                        device_id_type=MESH)
                    d.start()
                    rs_sends.append(d)

            for d in ag_sends:
                d.wait_send()

            acc = p_own
            for k in range(1, N_DEV):
                recv = pltpu.make_async_remote_copy(
                    src_ref=rs_ref.at[15 - k],
                    dst_ref=rs_ref.at[15 - k],
                    send_sem=ssem_rs.at[15 - k],
                    recv_sem=rsem_rs.at[15 - k],
                    device_id=((my + k) % N_DEV,),
                    device_id_type=MESH)
                recv.wait_recv()
                acc = acc + rs_ref[15 - k].astype(F32)

            if L < 2:
                X_ref[slot(0), :] = acc.astype(BF16)
                ag_sends = broadcast_own_chunk()
            else:
                out_ref[:] = acc
            for d in rs_sends:
                d.wait_send()

    return pl.pallas_call(
        body,
        out_shape=jax.ShapeDtypeStruct((B_PER, D), jnp.float32),
        in_specs=[pl.BlockSpec(memory_space=pltpu.VMEM)] * 7,
        out_specs=pl.BlockSpec(memory_space=pltpu.VMEM),
        scratch_shapes=[
            pltpu.VMEM((N_DEV * B_PER, D), BF16),
            pltpu.VMEM((N_DEV * B_PER, D), BF16),
            pltpu.VMEM((N_DEV - 1, B_PER, D), BF16),
            pltpu.SemaphoreType.DMA((N_DEV - 1,)),
            pltpu.SemaphoreType.DMA((N_DEV - 1,)),
            pltpu.SemaphoreType.DMA((N_DEV - 1,)),
            pltpu.SemaphoreType.DMA((N_DEV - 1,)),
        ],
        compiler_params=pltpu.CompilerParams(collective_id=0),
    )(x, Win0, Wout0, Win1, Wout1, Win2, Wout2)


# device time: 163989 ns/iter; 1.0161x vs baseline; 1.0161x over previous
import jax
import jax.numpy as jnp
from jax import lax
from jax.experimental import pallas as pl
from jax.experimental.pallas import tpu as pltpu

N_DEV = 16
B_PER = 256
D = 256
H_PER = 512
GROUP = 2
BF16 = jnp.bfloat16
F32 = jnp.float32
MESH = pl.DeviceIdType.MESH


def kernel(x, Win0, Wout0, Win1, Wout1, Win2, Wout2):
    def body(x_ref, win0_ref, wout0_ref, win1_ref, wout1_ref, win2_ref,
             wout2_ref, out_ref, X_ref, Pb_ref, rs_ref,
             ssem_ag, ssem_rs, rsem_ag, rsem_rs):
        my = lax.axis_index("i")

        def slot(k, n=1):
            return pl.ds(k * B_PER, n * B_PER)

        bsem = pltpu.get_barrier_semaphore()
        for k in range(1, N_DEV):
            pl.semaphore_signal(bsem, inc=1, device_id=((my + k) % N_DEV,),
                                device_id_type=MESH)
        pl.semaphore_wait(bsem, N_DEV - 1)

        def broadcast_own_chunk():
            sends = []
            for k in range(1, N_DEV):
                d = pltpu.make_async_remote_copy(
                    src_ref=X_ref.at[slot(0)],
                    dst_ref=X_ref.at[slot(k)],
                    send_sem=ssem_ag.at[k - 1], recv_sem=rsem_ag.at[k - 1],
                    device_id=((my + k) % N_DEV,), device_id_type=MESH)
                d.start()
                sends.append(d)
            return sends

        w_in = (win0_ref, win1_ref, win2_ref)
        w_out = (wout0_ref, wout1_ref, wout2_ref)

        X_ref[slot(0), :] = x_ref[:].astype(BF16)
        ag_sends = broadcast_own_chunk()

        for L in range(3):
            Wi = w_in[L][:].astype(BF16)
            Wo = w_out[L][:].astype(BF16)
            rs_sends = []
            p_own = None
            for g in range(N_DEV // GROUP):
                for k in range(g * GROUP, (g + 1) * GROUP):
                    if k == 0:
                        continue
                    recv = pltpu.make_async_remote_copy(
                        src_ref=X_ref.at[slot(k)],
                        dst_ref=X_ref.at[slot(k)],
                        send_sem=ssem_ag.at[k - 1],
                        recv_sem=rsem_ag.at[k - 1],
                        device_id=((my - k) % N_DEV,),
                        device_id_type=MESH)
                    recv.wait_recv()
                Xg = X_ref[slot(g * GROUP, GROUP), :]
                Hg = jnp.maximum(
                    jnp.dot(Xg, Wi, preferred_element_type=F32), 0.0)
                Pg = jnp.dot(Hg.astype(BF16), Wo,
                             preferred_element_type=F32)
                for j in range(GROUP):
                    k = g * GROUP + j
                    if k == 0:
                        p_own = Pg[:B_PER, :]
                        continue
                    Pb_ref[slot(k), :] = Pg[
                        j * B_PER:(j + 1) * B_PER, :].astype(BF16)
                    d = pltpu.make_async_remote_copy(
                        src_ref=Pb_ref.at[slot(k)],
                        dst_ref=rs_ref.at[15 - k],
                        send_sem=ssem_rs.at[15 - k],
                        recv_sem=rsem_rs.at[15 - k],
                        device_id=((my - k) % N_DEV,),
                        device_id_type=MESH)
                    d.start()
                    rs_sends.append(d)

            for d in ag_sends:
                d.wait_send()

            acc = p_own
            for k in range(1, N_DEV):
                recv = pltpu.make_async_remote_copy(
                    src_ref=rs_ref.at[15 - k],
                    dst_ref=rs_ref.at[15 - k],
                    send_sem=ssem_rs.at[15 - k],
                    recv_sem=rsem_rs.at[15 - k],
                    device_id=((my + k) % N_DEV,),
                    device_id_type=MESH)
                recv.wait_recv()
                acc = acc + rs_ref[15 - k].astype(F32)

            if L < 2:
                X_ref[slot(0), :] = acc.astype(BF16)
                ag_sends = broadcast_own_chunk()
            else:
                out_ref[:] = acc
            for d in rs_sends:
                d.wait_send()

    return pl.pallas_call(
        body,
        out_shape=jax.ShapeDtypeStruct((B_PER, D), jnp.float32),
        in_specs=[pl.BlockSpec(memory_space=pltpu.VMEM)] * 7,
        out_specs=pl.BlockSpec(memory_space=pltpu.VMEM),
        scratch_shapes=[
            pltpu.VMEM((N_DEV * B_PER, D), BF16),
            pltpu.VMEM((N_DEV * B_PER, D), BF16),
            pltpu.VMEM((N_DEV - 1, B_PER, D), BF16),
            pltpu.SemaphoreType.DMA((N_DEV - 1,)),
            pltpu.SemaphoreType.DMA((N_DEV - 1,)),
            pltpu.SemaphoreType.DMA((N_DEV - 1,)),
            pltpu.SemaphoreType.DMA((N_DEV - 1,)),
        ],
        compiler_params=pltpu.CompilerParams(collective_id=0),
    )(x, Win0, Wout0, Win1, Wout1, Win2, Wout2)
